# Initial kernel scaffold; baseline (speedup 1.0000x reference)
import jax
import jax.numpy as jnp
from jax.experimental import pallas as pl


def kernel(x, edge_index, target_node_index, W1, b1, W2, b2, W_ih, W_hh, b_ih, b_hh, fc_W, fc_b):
    raise NotImplementedError("write your pallas kernel here")


# speedup vs baseline: 7.0010x; 7.0010x over previous
import functools
import jax
import jax.numpy as jnp
from jax import lax
from jax.experimental import pallas as pl
from jax.experimental.pallas import tpu as pltpu
from jax.experimental.pallas import tpu_sc as plsc

NC = 2
NS = 16
NW = NC * NS
LN = 16

F32 = jnp.float32


def _sc_mesh():
    return plsc.VectorSubcoreMesh(core_axis_name="c", subcore_axis_name="s",
                                  num_cores=NC, num_subcores=NS)


def _make_deg_kernel(npad, n_chunk_rows):
    rows_per_tile = n_chunk_rows // NW
    zrows = npad // NS

    @functools.partial(
        pl.kernel,
        out_type=jax.ShapeDtypeStruct((NC * npad,), F32),
        mesh=_sc_mesh(),
        scratch_types=[
            pltpu.VMEM((rows_per_tile, 128), jnp.int32),
            pltpu.VMEM((128,), F32),
            pltpu.VMEM((zrows,), F32),
            pltpu.VMEM_SHARED((npad,), F32),
        ],
    )
    def deg_kernel(dst2d, out, idx_v, ones_v, zbuf, cnt_sh):
        c = lax.axis_index("c")
        s = lax.axis_index("s")
        wid = s * NC + c

        def zb(i, carry):
            zbuf[pl.ds(i * LN, LN)] = jnp.zeros((LN,), F32)
            return carry
        lax.fori_loop(0, zrows // LN, zb, 0)
        for k in range(128 // LN):
            ones_v[pl.ds(k * LN, LN)] = jnp.ones((LN,), F32)
        pltpu.sync_copy(zbuf, cnt_sh.at[pl.ds(s * zrows, zrows)])
        plsc.subcore_barrier()

        pltpu.sync_copy(dst2d.at[pl.ds(wid * rows_per_tile, rows_per_tile)], idx_v)

        def body(j, carry):
            pltpu.sync_copy(ones_v, cnt_sh.at[idx_v.at[j]], add=True)
            return carry
        lax.fori_loop(0, rows_per_tile, body, 0)

        plsc.subcore_barrier()
        pltpu.sync_copy(cnt_sh.at[pl.ds(s * zrows, zrows)],
                        out.at[pl.ds(c * npad + s * zrows, zrows)])

    return deg_kernel


def _make_agg_kernel(npad, n_chunk_rows):
    rows_per_tile = n_chunk_rows // NS
    zrows = npad // NS

    @functools.partial(
        pl.kernel,
        out_type=jax.ShapeDtypeStruct((NC * npad, 128), F32),
        mesh=_sc_mesh(),
        scratch_types=[
            pltpu.VMEM((2, 128), jnp.int32),
            pltpu.VMEM((2, 128), jnp.int32),
            pltpu.VMEM((128, 128), F32),
            pltpu.VMEM((64, 128), F32),
            pltpu.VMEM_SHARED((npad, 128), F32),
            pltpu.SemaphoreType.DMA,
            pltpu.SemaphoreType.DMA,
        ],
    )
    def agg_kernel(table, src2d, dst2d, out, sidx, didx, rows, zbuf, acc_sh,
                   sem, isem):
        c = lax.axis_index("c")
        s = lax.axis_index("s")

        def zb(i, carry):
            for k in range(128 // LN):
                zbuf[i, pl.ds(k * LN, LN)] = jnp.zeros((LN,), F32)
            return carry
        lax.fori_loop(0, 64, zb, 0)
        for r in range(zrows // 64):
            pltpu.sync_copy(zbuf, acc_sh.at[pl.ds(s * zrows + r * 64, 64)])
        plsc.subcore_barrier()

        base = s * rows_per_tile
        off = c * npad

        def fetch(j, p):
            pltpu.make_async_copy(src2d.at[pl.ds(base + j, 1)],
                                  sidx.at[pl.ds(p, 1)], isem).start()
            pltpu.make_async_copy(dst2d.at[pl.ds(base + j, 1)],
                                  didx.at[pl.ds(p, 1)], isem).start()

        def drain_idx():
            pltpu.make_async_copy(src2d.at[pl.ds(0, 1)],
                                  sidx.at[pl.ds(0, 1)], isem).wait()
            pltpu.make_async_copy(dst2d.at[pl.ds(0, 1)],
                                  didx.at[pl.ds(0, 1)], isem).wait()

        fetch(0, 0)

        def body(jj, carry):
            for p in range(2):
                j = jj * 2 + p
                drain_idx()
                @pl.when(j + 1 < rows_per_tile)
                def _():
                    fetch(j + 1, 1 - p)
                for k in range(128 // LN):
                    sl = pl.ds(k * LN, LN)
                    sidx[p, sl] = sidx[p, sl] + off
                pltpu.async_copy(table.at[sidx.at[p]], rows, sem).wait()
                pltpu.sync_copy(rows, acc_sh.at[didx.at[p]], add=True)
            return carry
        lax.fori_loop(0, rows_per_tile // 2, body, 0)

        plsc.subcore_barrier()
        for r in range(zrows // 128):
            pltpu.sync_copy(acc_sh.at[pl.ds(s * zrows + r * 128, 128)],
                            out.at[pl.ds(c * npad + s * zrows + r * 128, 128)])

    return agg_kernel


def _make_tgather_kernel(npad, b):
    bpw = b // NW

    @functools.partial(
        pl.kernel,
        out_type=(
            jax.ShapeDtypeStruct((2, b, 128), F32),
            jax.ShapeDtypeStruct((2, b, 128), F32),
            jax.ShapeDtypeStruct((b,), F32),
        ),
        mesh=_sc_mesh(),
        scratch_types=[
            pltpu.VMEM((bpw,), jnp.int32),
            pltpu.VMEM((bpw,), jnp.int32),
            pltpu.VMEM((bpw, 128), F32),
            pltpu.VMEM((bpw, 128), F32),
            pltpu.VMEM((bpw, 128), F32),
            pltpu.VMEM((bpw, 128), F32),
            pltpu.VMEM((bpw,), F32),
            pltpu.SemaphoreType.DMA,
        ],
    )
    def tg_kernel(acc_t, hs_t, dinv_t, tgt, gacc, ghs, gdinv,
                  tidx, tidx_hi, ra, rb, rc, rd, dv, sem):
        c = lax.axis_index("c")
        s = lax.axis_index("s")
        wid = s * NC + c
        base = wid * bpw

        pltpu.sync_copy(tgt.at[pl.ds(base, bpw)], tidx)
        for k in range(bpw // LN):
            sl = pl.ds(k * LN, LN)
            tidx_hi[sl] = tidx[sl] + npad

        pltpu.async_copy(acc_t.at[tidx], ra, sem).wait()
        pltpu.async_copy(acc_t.at[tidx_hi], rb, sem).wait()
        pltpu.async_copy(hs_t.at[tidx], rc, sem).wait()
        pltpu.async_copy(hs_t.at[tidx_hi], rd, sem).wait()
        pltpu.async_copy(dinv_t.at[tidx], dv, sem).wait()

        pltpu.sync_copy(ra, gacc.at[0, pl.ds(base, bpw)])
        pltpu.sync_copy(rb, gacc.at[1, pl.ds(base, bpw)])
        pltpu.sync_copy(rc, ghs.at[0, pl.ds(base, bpw)])
        pltpu.sync_copy(rd, ghs.at[1, pl.ds(base, bpw)])
        pltpu.sync_copy(dv, gdinv.at[pl.ds(base, bpw)])

    return tg_kernel


def _mm_scale_body(x_ref, w_ref, ca_ref, cb_ref, hs_ref, dinv_ref):
    deg = ca_ref[...] + cb_ref[...] + 1.0
    dv = lax.rsqrt(deg)
    h = jnp.dot(x_ref[...], w_ref[...], preferred_element_type=F32)
    hs_ref[0] = dv * h
    dinv_ref[...] = dv


def _make_mm_scale(npad, d, blk):
    nb = npad // blk
    return pl.pallas_call(
        _mm_scale_body,
        grid=(nb, 2),
        in_specs=[
            pl.BlockSpec((blk, d), lambda i, c: (i, 0)),
            pl.BlockSpec((d, 128), lambda i, c: (0, c)),
            pl.BlockSpec((blk, 1), lambda i, c: (i, 0)),
            pl.BlockSpec((blk, 1), lambda i, c: (i, 0)),
        ],
        out_specs=[
            pl.BlockSpec((1, blk, 128), lambda i, c: (c, i, 0)),
            pl.BlockSpec((blk, 1), lambda i, c: (i, 0)),
        ],
        out_shape=[
            jax.ShapeDtypeStruct((2, npad, 128), F32),
            jax.ShapeDtypeStruct((npad, 1), F32),
        ],
    )


def _ew_body(acc_ref, hs_ref, dinv_ref, b_ref, out_ref):
    dv = dinv_ref[...]
    a = acc_ref[...] + hs_ref[...]
    h1 = jnp.maximum(dv * a + b_ref[0], 0.0)
    out_ref[...] = dv * h1


def _make_ew(npad, blk):
    nb = npad // blk
    return pl.pallas_call(
        _ew_body,
        grid=(2, nb),
        in_specs=[
            pl.BlockSpec((blk, 128), lambda c, i: (c * nb + i, 0)),
            pl.BlockSpec((blk, 128), lambda c, i: (c * nb + i, 0)),
            pl.BlockSpec((blk, 1), lambda c, i: (i, 0)),
            pl.BlockSpec((1, 1, 128), lambda c, i: (c, 0, 0)),
        ],
        out_specs=pl.BlockSpec((blk, 128), lambda c, i: (c * nb + i, 0)),
        out_shape=jax.ShapeDtypeStruct((2 * npad, 128), F32),
    )


def _head_body(gacc_ref, ghs_ref, gdinv_ref, w2t_ref, b2_ref, wih_ref,
               bih_ref, bhh_ref, fcw_ref, fcb_ref, out_ref):
    ga = gacc_ref[...]
    gh = ghs_ref[...]
    gsum = jnp.concatenate([ga[0] + gh[0], ga[1] + gh[1]], axis=1)
    tpre = gdinv_ref[...] * gsum
    t = jnp.maximum(jnp.dot(tpre, w2t_ref[...], preferred_element_type=F32)
                    + b2_ref[...], 0.0)
    gi = jnp.dot(t, wih_ref[...], preferred_element_type=F32) + bih_ref[...]
    bhh = bhh_ref[...]
    gh_dim = t.shape[1]
    i_r = gi[:, :gh_dim]
    i_z = gi[:, gh_dim:2 * gh_dim]
    i_n = gi[:, 2 * gh_dim:]
    h_r = bhh[:, :gh_dim]
    h_z = bhh[:, gh_dim:2 * gh_dim]
    h_n = bhh[:, 2 * gh_dim:]
    r = jax.nn.sigmoid(i_r + h_r)
    z = jax.nn.sigmoid(i_z + h_z)
    n_ = jnp.tanh(i_n + r * h_n)
    hN = (1.0 - z) * n_
    out_ref[...] = jnp.dot(hN, fcw_ref[...], preferred_element_type=F32) + fcb_ref[...]


def _make_head(b, h):
    return pl.pallas_call(
        _head_body,
        out_shape=jax.ShapeDtypeStruct((b, 128), F32),
    )


def kernel(x, edge_index, target_node_index, W1, b1, W2, b2,
           W_ih, W_hh, b_ih, b_hh, fc_W, fc_b):
    n, d = x.shape
    e = edge_index.shape[1]
    b = target_node_index.shape[0]
    h = W1.shape[0]
    c_out = fc_W.shape[0]

    npad = ((n + 1023) // 1024) * 1024
    dump = n
    epad = ((e + NW * 128 - 1) // (NW * 128)) * (NW * 128)
    n_chunk_rows = epad // 128

    i32 = jnp.int32
    src = edge_index[0]
    dst = edge_index[1]
    padlen = epad - e
    src2d = jnp.concatenate(
        [src, jnp.full((padlen,), dump, i32)]).reshape(n_chunk_rows, 128)
    dst2d = jnp.concatenate(
        [dst, jnp.full((padlen,), dump, i32)]).reshape(n_chunk_rows, 128)

    x_pad = jnp.pad(x, ((0, npad - n), (0, 0)))
    w1t = W1.T
    w2t = W2.T
    wih_t = W_ih.T
    fcw_t = jnp.pad(fc_W.T, ((0, 0), (0, 128 - c_out)))
    fcb_p = jnp.pad(fc_b, (0, 128 - c_out)).reshape(1, 128)
    b1r = b1.reshape(2, 1, 128)
    b2r = b2.reshape(1, h)
    bihr = b_ih.reshape(1, 3 * h)
    bhhr = b_hh.reshape(1, 3 * h)

    cnt = _make_deg_kernel(npad, n_chunk_rows)(dst2d)
    ca = cnt[:npad].reshape(npad, 1)
    cb = cnt[npad:].reshape(npad, 1)

    hs3, dinv = _make_mm_scale(npad, d, 512)(x_pad, w1t, ca, cb)
    hs = hs3.reshape(2 * npad, 128)

    agg = _make_agg_kernel(npad, n_chunk_rows)
    acc1 = agg(hs, src2d, dst2d)

    hs2 = _make_ew(npad, 512)(acc1, hs, dinv, b1r)

    acc2 = agg(hs2, src2d, dst2d)

    gacc, ghs, gdinv = _make_tgather_kernel(npad, b)(
        acc2, hs2, dinv.reshape(npad), target_node_index)

    out128 = _make_head(b, h)(gacc, ghs, gdinv.reshape(b, 1), w2t, b2r,
                              wih_t, bihr, bhhr, fcw_t, fcb_p)
    return out128[:, :c_out]


# speedup vs baseline: 8.3797x; 1.1969x over previous
import functools
import jax
import jax.numpy as jnp
from jax import lax
from jax.experimental import pallas as pl
from jax.experimental.pallas import tpu as pltpu
from jax.experimental.pallas import tpu_sc as plsc

NC = 2
NS = 16
NW = NC * NS
LN = 16

F32 = jnp.float32


def _sc_mesh():
    return plsc.VectorSubcoreMesh(core_axis_name="c", subcore_axis_name="s",
                                  num_cores=NC, num_subcores=NS)


CH = 96


def _make_deg_kernel(npad, n_chunk_rows):
    rows_per_tile = n_chunk_rows // NW
    zrows = npad // NS

    @functools.partial(
        pl.kernel,
        out_type=jax.ShapeDtypeStruct((NC * npad,), F32),
        mesh=_sc_mesh(),
        scratch_types=[
            pltpu.VMEM((rows_per_tile, 1, CH), jnp.int32),
            pltpu.VMEM((CH,), F32),
            pltpu.VMEM((zrows,), F32),
            pltpu.VMEM_SHARED((npad,), F32),
        ],
    )
    def deg_kernel(dst3, out, idx_v, ones_v, zbuf, cnt_sh):
        c = lax.axis_index("c")
        s = lax.axis_index("s")
        wid = s * NC + c

        def zb(i, carry):
            zbuf[pl.ds(i * LN, LN)] = jnp.zeros((LN,), F32)
            return carry
        lax.fori_loop(0, zrows // LN, zb, 0)
        for k in range(CH // LN):
            ones_v[pl.ds(k * LN, LN)] = jnp.ones((LN,), F32)
        pltpu.sync_copy(zbuf, cnt_sh.at[pl.ds(s * zrows, zrows)])
        plsc.subcore_barrier()

        pltpu.sync_copy(dst3.at[pl.ds(wid * rows_per_tile, rows_per_tile)], idx_v)

        def body(j, carry):
            pltpu.sync_copy(ones_v, cnt_sh.at[idx_v.at[j, 0]], add=True)
            return carry
        lax.fori_loop(0, rows_per_tile, body, 0)

        plsc.subcore_barrier()
        pltpu.sync_copy(cnt_sh.at[pl.ds(s * zrows, zrows)],
                        out.at[pl.ds(c * npad + s * zrows, zrows)])

    return deg_kernel


def _make_agg_kernel(npad, n_chunk_rows):
    rows_per_tile = n_chunk_rows // NS
    zrows = npad // NS

    @functools.partial(
        pl.kernel,
        out_type=jax.ShapeDtypeStruct((NC * npad, 128), F32),
        mesh=_sc_mesh(),
        scratch_types=[
            pltpu.VMEM((2, 1, CH), jnp.int32),
            pltpu.VMEM((2, 1, CH), jnp.int32),
            pltpu.VMEM((2, CH, 128), F32),
            pltpu.VMEM_SHARED((npad, 128), F32),
            pltpu.SemaphoreType.DMA,
            pltpu.SemaphoreType.DMA,
        ],
    )
    def agg_kernel(table, src3, dst3, out, sidx, didx, rows, acc_sh,
                   gsem, isem):
        c = lax.axis_index("c")
        s = lax.axis_index("s")

        def zb(i, carry):
            for k in range(128 // LN):
                rows[0, i, pl.ds(k * LN, LN)] = jnp.zeros((LN,), F32)
            return carry
        lax.fori_loop(0, CH, zb, 0)
        for r in range(zrows // CH):
            pltpu.sync_copy(rows.at[0], acc_sh.at[pl.ds(s * zrows + r * CH, CH)])
        plsc.subcore_barrier()

        base = s * rows_per_tile
        off = c * npad

        def fetch(j, p):
            pltpu.make_async_copy(src3.at[pl.ds(base + j, 1)],
                                  sidx.at[pl.ds(p, 1)], isem).start()
            pltpu.make_async_copy(dst3.at[pl.ds(base + j, 1)],
                                  didx.at[pl.ds(p, 1)], isem).start()

        def drain_idx():
            pltpu.make_async_copy(src3.at[pl.ds(0, 1)],
                                  sidx.at[pl.ds(0, 1)], isem).wait()
            pltpu.make_async_copy(dst3.at[pl.ds(0, 1)],
                                  didx.at[pl.ds(0, 1)], isem).wait()

        def shift(p):
            for k in range(CH // LN):
                sl = pl.ds(k * LN, LN)
                sidx[p, 0, sl] = sidx[p, 0, sl] + off

        def start_gather(p):
            pltpu.make_async_copy(table.at[sidx.at[p, 0]], rows.at[p],
                                  gsem).start()

        def wait_gather(p):
            pltpu.make_async_copy(table.at[pl.ds(0, CH)], rows.at[p],
                                  gsem).wait()

        fetch(0, 0)
        drain_idx()
        shift(0)
        start_gather(0)
        fetch(1, 1)

        def body(jj, carry):
            for p in range(2):
                j = jj * 2 + p
                wait_gather(p)

                @pl.when(j + 1 < rows_per_tile)
                def _():
                    drain_idx()
                    shift(1 - p)
                    start_gather(1 - p)
                pltpu.sync_copy(rows.at[p], acc_sh.at[didx.at[p, 0]], add=True)

                @pl.when(j + 2 < rows_per_tile)
                def _():
                    fetch(j + 2, p)
            return carry
        lax.fori_loop(0, rows_per_tile // 2, body, 0)

        plsc.subcore_barrier()
        for r in range(zrows // 128):
            pltpu.sync_copy(acc_sh.at[pl.ds(s * zrows + r * 128, 128)],
                            out.at[pl.ds(c * npad + s * zrows + r * 128, 128)])

    return agg_kernel


def _make_tgather_kernel(npad, b):
    bpw = b // NW

    @functools.partial(
        pl.kernel,
        out_type=(
            jax.ShapeDtypeStruct((2, b, 128), F32),
            jax.ShapeDtypeStruct((2, b, 128), F32),
            jax.ShapeDtypeStruct((b,), F32),
        ),
        mesh=_sc_mesh(),
        scratch_types=[
            pltpu.VMEM((bpw,), jnp.int32),
            pltpu.VMEM((bpw,), jnp.int32),
            pltpu.VMEM((bpw, 128), F32),
            pltpu.VMEM((bpw, 128), F32),
            pltpu.VMEM((bpw, 128), F32),
            pltpu.VMEM((bpw, 128), F32),
            pltpu.VMEM((bpw,), F32),
            pltpu.SemaphoreType.DMA,
        ],
    )
    def tg_kernel(acc_t, hs_t, dinv_t, tgt, gacc, ghs, gdinv,
                  tidx, tidx_hi, ra, rb, rc, rd, dv, sem):
        c = lax.axis_index("c")
        s = lax.axis_index("s")
        wid = s * NC + c
        base = wid * bpw

        pltpu.sync_copy(tgt.at[pl.ds(base, bpw)], tidx)
        for k in range(bpw // LN):
            sl = pl.ds(k * LN, LN)
            tidx_hi[sl] = tidx[sl] + npad

        pltpu.async_copy(acc_t.at[tidx], ra, sem).wait()
        pltpu.async_copy(acc_t.at[tidx_hi], rb, sem).wait()
        pltpu.async_copy(hs_t.at[tidx], rc, sem).wait()
        pltpu.async_copy(hs_t.at[tidx_hi], rd, sem).wait()
        pltpu.async_copy(dinv_t.at[tidx], dv, sem).wait()

        pltpu.sync_copy(ra, gacc.at[0, pl.ds(base, bpw)])
        pltpu.sync_copy(rb, gacc.at[1, pl.ds(base, bpw)])
        pltpu.sync_copy(rc, ghs.at[0, pl.ds(base, bpw)])
        pltpu.sync_copy(rd, ghs.at[1, pl.ds(base, bpw)])
        pltpu.sync_copy(dv, gdinv.at[pl.ds(base, bpw)])

    return tg_kernel


def _mm_scale_body(x_ref, w_ref, ca_ref, cb_ref, hs_ref, dinv_ref):
    deg = ca_ref[...] + cb_ref[...] + 1.0
    dv = lax.rsqrt(deg)
    h = jnp.dot(x_ref[...], w_ref[...], preferred_element_type=F32)
    hs_ref[0] = dv * h
    dinv_ref[...] = dv


def _make_mm_scale(npad, d, blk):
    nb = npad // blk
    return pl.pallas_call(
        _mm_scale_body,
        grid=(nb, 2),
        in_specs=[
            pl.BlockSpec((blk, d), lambda i, c: (i, 0)),
            pl.BlockSpec((d, 128), lambda i, c: (0, c)),
            pl.BlockSpec((blk, 1), lambda i, c: (i, 0)),
            pl.BlockSpec((blk, 1), lambda i, c: (i, 0)),
        ],
        out_specs=[
            pl.BlockSpec((1, blk, 128), lambda i, c: (c, i, 0)),
            pl.BlockSpec((blk, 1), lambda i, c: (i, 0)),
        ],
        out_shape=[
            jax.ShapeDtypeStruct((2, npad, 128), F32),
            jax.ShapeDtypeStruct((npad, 1), F32),
        ],
    )


def _ew_body(acc_ref, hs_ref, dinv_ref, b_ref, out_ref):
    dv = dinv_ref[...]
    a = acc_ref[...] + hs_ref[...]
    h1 = jnp.maximum(dv * a + b_ref[0], 0.0)
    out_ref[...] = dv * h1


def _make_ew(npad, blk):
    nb = npad // blk
    return pl.pallas_call(
        _ew_body,
        grid=(2, nb),
        in_specs=[
            pl.BlockSpec((blk, 128), lambda c, i: (c * nb + i, 0)),
            pl.BlockSpec((blk, 128), lambda c, i: (c * nb + i, 0)),
            pl.BlockSpec((blk, 1), lambda c, i: (i, 0)),
            pl.BlockSpec((1, 1, 128), lambda c, i: (c, 0, 0)),
        ],
        out_specs=pl.BlockSpec((blk, 128), lambda c, i: (c * nb + i, 0)),
        out_shape=jax.ShapeDtypeStruct((2 * npad, 128), F32),
    )


def _head_body(gacc_ref, ghs_ref, gdinv_ref, w2t_ref, b2_ref, wih_ref,
               bih_ref, bhh_ref, fcw_ref, fcb_ref, out_ref):
    ga = gacc_ref[...]
    gh = ghs_ref[...]
    gsum = jnp.concatenate([ga[0] + gh[0], ga[1] + gh[1]], axis=1)
    tpre = gdinv_ref[...] * gsum
    t = jnp.maximum(jnp.dot(tpre, w2t_ref[...], preferred_element_type=F32)
                    + b2_ref[...], 0.0)
    gi = jnp.dot(t, wih_ref[...], preferred_element_type=F32) + bih_ref[...]
    bhh = bhh_ref[...]
    gh_dim = t.shape[1]
    i_r = gi[:, :gh_dim]
    i_z = gi[:, gh_dim:2 * gh_dim]
    i_n = gi[:, 2 * gh_dim:]
    h_r = bhh[:, :gh_dim]
    h_z = bhh[:, gh_dim:2 * gh_dim]
    h_n = bhh[:, 2 * gh_dim:]
    r = jax.nn.sigmoid(i_r + h_r)
    z = jax.nn.sigmoid(i_z + h_z)
    n_ = jnp.tanh(i_n + r * h_n)
    hN = (1.0 - z) * n_
    out_ref[...] = jnp.dot(hN, fcw_ref[...], preferred_element_type=F32) + fcb_ref[...]


def _make_head(b, h):
    return pl.pallas_call(
        _head_body,
        out_shape=jax.ShapeDtypeStruct((b, 128), F32),
    )


def kernel(x, edge_index, target_node_index, W1, b1, W2, b2,
           W_ih, W_hh, b_ih, b_hh, fc_W, fc_b):
    n, d = x.shape
    e = edge_index.shape[1]
    b = target_node_index.shape[0]
    h = W1.shape[0]
    c_out = fc_W.shape[0]

    npad = ((n + 1 + 4095) // 4096) * 4096
    dump = n
    epad = ((e + NW * CH - 1) // (NW * CH)) * (NW * CH)
    n_chunk_rows = epad // CH

    i32 = jnp.int32
    src = edge_index[0]
    dst = edge_index[1]
    padlen = epad - e
    src3 = jnp.concatenate(
        [src, jnp.full((padlen,), dump, i32)]).reshape(n_chunk_rows, 1, CH)
    dst3 = jnp.concatenate(
        [dst, jnp.full((padlen,), dump, i32)]).reshape(n_chunk_rows, 1, CH)

    x_pad = jnp.pad(x, ((0, npad - n), (0, 0)))
    w1t = W1.T
    w2t = W2.T
    wih_t = W_ih.T
    fcw_t = jnp.pad(fc_W.T, ((0, 0), (0, 128 - c_out)))
    fcb_p = jnp.pad(fc_b, (0, 128 - c_out)).reshape(1, 128)
    b1r = b1.reshape(2, 1, 128)
    b2r = b2.reshape(1, h)
    bihr = b_ih.reshape(1, 3 * h)
    bhhr = b_hh.reshape(1, 3 * h)

    cnt = _make_deg_kernel(npad, n_chunk_rows)(dst3)
    ca = cnt[:npad].reshape(npad, 1)
    cb = cnt[npad:].reshape(npad, 1)

    hs3, dinv = _make_mm_scale(npad, d, 512)(x_pad, w1t, ca, cb)
    hs = hs3.reshape(2 * npad, 128)

    agg = _make_agg_kernel(npad, n_chunk_rows)
    acc1 = agg(hs, src3, dst3)

    hs2 = _make_ew(npad, 512)(acc1, hs, dinv, b1r)

    acc2 = agg(hs2, src3, dst3)

    gacc, ghs, gdinv = _make_tgather_kernel(npad, b)(
        acc2, hs2, dinv.reshape(npad), target_node_index)

    out128 = _make_head(b, h)(gacc, ghs, gdinv.reshape(b, 1), w2t, b2r,
                              wih_t, bihr, bhhr, fcw_t, fcb_p)
    return out128[:, :c_out]
